# baseline (speedup 1.0000x reference)
import functools

import numpy as np
import jax
import jax.numpy as jnp
from jax import lax
from jax.experimental import pallas as pl
from jax.experimental.pallas import tpu as pltpu
from jax.experimental.pallas import tpu_sc as plsc

B = 4096
S = 200
V = 100000
H = 128
L = 16
NC = 2
NS = 16
NW = NC * NS
BPW = B // NW
IPW = BPW * S
GW = 128
NSLICE = IPW // GW

ACC_ROWS = NS * BPW

CK = 10
NCHUNK = NSLICE // CK
NPAIR = NCHUNK // 2

_DST = (
    np.arange(NS, dtype=np.int32)[:, None] * BPW
    + (np.arange(IPW, dtype=np.int32) // S)[None, :]
).reshape(NS * NCHUNK, CK * GW)

VP = V // 8
KP = 8 * H
NP = 8 * L
_PROJ_BLK = 1000
_PROJ_GRID = -(-VP // _PROJ_BLK)


def _project_body(tab_ref, whi_ref, wlo_ref, b_ref, o_ref):
    del wlo_ref
    o_ref[...] = (
        jnp.dot(tab_ref[...], whi_ref[...], preferred_element_type=jnp.float32,
                precision=lax.Precision.HIGHEST)
        + b_ref[...]
    )


def _project(table8, whi, wlo, bblk):
    return pl.pallas_call(
        _project_body,
        grid=(_PROJ_GRID,),
        in_specs=[
            pl.BlockSpec((_PROJ_BLK, KP), lambda i: (i, 0)),
            pl.BlockSpec((KP, NP), lambda i: (0, 0)),
            pl.BlockSpec((KP, NP), lambda i: (0, 0)),
            pl.BlockSpec((1, NP), lambda i: (0, 0)),
        ],
        out_specs=pl.BlockSpec((_PROJ_BLK, NP), lambda i: (i, 0)),
        out_shape=jax.ShapeDtypeStruct((VP, NP), jnp.float32),
    )(table8, whi, wlo, bblk)


def _pool(proj, ids2d, dst2d):
    mesh = plsc.VectorSubcoreMesh(core_axis_name="c", subcore_axis_name="s")

    @functools.partial(
        pl.kernel,
        out_type=jax.ShapeDtypeStruct((B, L), jnp.float32),
        mesh=mesh,
        scratch_types=[
            pltpu.VMEM((NCHUNK, CK * GW), jnp.int32),
            pltpu.VMEM((NCHUNK, CK * GW), jnp.int32),
            pltpu.VMEM((CK * GW, L), jnp.float32),
            pltpu.VMEM((CK * GW, L), jnp.float32),
            pltpu.VMEM_SHARED((ACC_ROWS, L), jnp.float32),
            pltpu.SemaphoreType.DMA,
            pltpu.SemaphoreType.DMA,
        ],
        compiler_params=pltpu.CompilerParams(use_tc_tiling_on_sc=False),
    )
    def k(proj_hbm, ids_hbm, dst_hbm, out_hbm, idx_v, dst_v, rows_a, rows_b,
          acc_sh, sem_a, sem_b):
        c = lax.axis_index("c")
        s = lax.axis_index("s")
        wid = c * NS + s

        pltpu.sync_copy(ids_hbm.at[pl.ds(wid * NCHUNK, NCHUNK)], idx_v)
        pltpu.sync_copy(dst_hbm.at[pl.ds(s * NCHUNK, NCHUNK)], dst_v)

        @pl.loop(0, BPW)
        def _zero(i):
            rows_a[i] = jnp.zeros((L,), jnp.float32)

        pltpu.sync_copy(rows_a.at[pl.ds(0, BPW)], acc_sh.at[pl.ds(s * BPW, BPW)])

        def g_start(ch, buf, sem):
            pltpu.async_copy(proj_hbm.at[idx_v.at[ch]], buf, sem)

        def g_wait(ch, buf, sem):
            pltpu.make_async_copy(proj_hbm.at[idx_v.at[ch]], buf, sem).wait()

        def scat(ch, buf):
            pltpu.sync_copy(buf, acc_sh.at[dst_v.at[ch]], add=True)

        g_start(0, rows_a, sem_a)

        @pl.loop(0, NPAIR)
        def _pair(t):
            c0 = 2 * t
            c1 = c0 + 1
            g_wait(c0, rows_a, sem_a)
            g_start(c1, rows_b, sem_b)
            scat(c0, rows_a)
            g_wait(c1, rows_b, sem_b)

            @pl.when(t < NPAIR - 1)
            def _():
                g_start(c0 + 2, rows_a, sem_a)

            scat(c1, rows_b)

        pltpu.sync_copy(acc_sh.at[pl.ds(s * BPW, BPW)], out_hbm.at[pl.ds(wid * BPW, BPW)])

    return k(proj, ids2d, dst2d)


def kernel(input_ids, attention_mask, embedding_table, classifier_w, classifier_b):
    del attention_mask
    ids2d = input_ids.reshape(NW * NCHUNK, CK * GW).astype(jnp.int32)
    dst2d = jnp.asarray(_DST)
    scale = jnp.float32(1.0 / S)
    wp = classifier_w.T * scale
    wblk = jnp.zeros((8, H, 8, L), jnp.float32)
    for i in range(8):
        wblk = wblk.at[i, :, i, :2].set(wp)
    wblk = wblk.reshape(KP, NP)
    whi = wblk
    wlo = wblk
    bblk = jnp.tile(
        jnp.zeros((L,), jnp.float32).at[:2].set(classifier_b * scale), 8
    ).reshape(1, NP)
    table8 = embedding_table.reshape(VP, KP)
    proj = _project(table8, whi, wlo, bblk).reshape(V, L)
    pooled = _pool(proj, ids2d, dst2d)
    return pooled[:, :2]


# speedup vs baseline: 1.1504x; 1.1504x over previous
import functools

import numpy as np
import jax
import jax.numpy as jnp
from jax import lax
from jax.experimental import pallas as pl
from jax.experimental.pallas import tpu as pltpu
from jax.experimental.pallas import tpu_sc as plsc

B = 4096
S = 200
V = 100000
H = 128
L = 16
NC = 2
NS = 16
NW = NC * NS
BPW = B // NW
IPW = BPW * S
GW = 128
NSLICE = IPW // GW

RPC = 8
CHUNK = RPC * S
NCHUNK = BPW // RPC
NPAIR = NCHUNK // 2

VP = V // 8
KP = 8 * H
NP = 8 * L
_PROJ_BLK = 1000
_PROJ_GRID = -(-VP // _PROJ_BLK)


def _project_body(tab_ref, whi_ref, wlo_ref, b_ref, o_ref):
    del wlo_ref
    o_ref[...] = (
        jnp.dot(tab_ref[...], whi_ref[...], preferred_element_type=jnp.float32,
                precision=lax.Precision.HIGHEST)
        + b_ref[...]
    )


def _project(table8, whi, wlo, bblk):
    return pl.pallas_call(
        _project_body,
        grid=(_PROJ_GRID,),
        in_specs=[
            pl.BlockSpec((_PROJ_BLK, KP), lambda i: (i, 0)),
            pl.BlockSpec((KP, NP), lambda i: (0, 0)),
            pl.BlockSpec((KP, NP), lambda i: (0, 0)),
            pl.BlockSpec((1, NP), lambda i: (0, 0)),
        ],
        out_specs=pl.BlockSpec((_PROJ_BLK, NP), lambda i: (i, 0)),
        out_shape=jax.ShapeDtypeStruct((VP, NP), jnp.float32),
    )(table8, whi, wlo, bblk)


def _pool(proj, ids2d):
    mesh = plsc.VectorSubcoreMesh(core_axis_name="c", subcore_axis_name="s")

    @functools.partial(
        pl.kernel,
        out_type=jax.ShapeDtypeStruct((B, L), jnp.float32),
        mesh=mesh,
        scratch_types=[
            pltpu.VMEM((NCHUNK, CHUNK), jnp.int32),
            pltpu.VMEM((CHUNK, L), jnp.float32),
            pltpu.VMEM((CHUNK, L), jnp.float32),
            pltpu.VMEM((BPW, L), jnp.float32),
            pltpu.SemaphoreType.DMA,
            pltpu.SemaphoreType.DMA,
        ],
        compiler_params=pltpu.CompilerParams(use_tc_tiling_on_sc=False),
    )
    def k(proj_hbm, ids_hbm, out_hbm, idx_v, rows_a, rows_b, out_v, sem_a, sem_b):
        c = lax.axis_index("c")
        s = lax.axis_index("s")
        wid = c * NS + s

        pltpu.sync_copy(ids_hbm.at[pl.ds(wid * NCHUNK, NCHUNK)], idx_v)

        def g_start(ch, buf, sem):
            pltpu.async_copy(proj_hbm.at[idx_v.at[ch]], buf, sem)

        def g_wait(ch, buf, sem):
            pltpu.make_async_copy(proj_hbm.at[idx_v.at[ch]], buf, sem).wait()

        def reduce(ch, buf):
            for b in range(RPC):
                def body(i, a):
                    return a + buf[b * S + i]
                acc = lax.fori_loop(0, S, body, jnp.zeros((L,), jnp.float32),
                                    unroll=8)
                out_v[ch * RPC + b] = acc

        g_start(0, rows_a, sem_a)

        @pl.loop(0, NPAIR)
        def _pair(t):
            c0 = 2 * t
            c1 = c0 + 1
            g_wait(c0, rows_a, sem_a)
            g_start(c1, rows_b, sem_b)
            reduce(c0, rows_a)
            g_wait(c1, rows_b, sem_b)

            @pl.when(t < NPAIR - 1)
            def _():
                g_start(c0 + 2, rows_a, sem_a)

            reduce(c1, rows_b)

        pltpu.sync_copy(out_v, out_hbm.at[pl.ds(wid * BPW, BPW)])

    return k(proj, ids2d)


def kernel(input_ids, attention_mask, embedding_table, classifier_w, classifier_b):
    del attention_mask
    ids2d = input_ids.reshape(NW * NCHUNK, CHUNK).astype(jnp.int32)
    scale = jnp.float32(1.0 / S)
    wp = classifier_w.T * scale
    wblk = jnp.zeros((8, H, 8, L), jnp.float32)
    for i in range(8):
        wblk = wblk.at[i, :, i, :2].set(wp)
    wblk = wblk.reshape(KP, NP)
    whi = wblk
    wlo = wblk
    bblk = jnp.tile(
        jnp.zeros((L,), jnp.float32).at[:2].set(classifier_b * scale), 8
    ).reshape(1, NP)
    table8 = embedding_table.reshape(VP, KP)
    proj = _project(table8, whi, wlo, bblk).reshape(V, L)
    pooled = _pool(proj, ids2d)
    return pooled[:, :2]


# speedup vs baseline: 1.4804x; 1.2869x over previous
import functools

import numpy as np
import jax
import jax.numpy as jnp
from jax import lax
from jax.experimental import pallas as pl
from jax.experimental.pallas import tpu as pltpu
from jax.experimental.pallas import tpu_sc as plsc

B = 4096
S = 200
V = 100000
H = 128
L = 16
NC = 2
NS = 16
NW = NC * NS
BPW = B // NW
IPW = BPW * S
GW = 128
NSLICE = IPW // GW

RPC = 8
CHUNK = RPC * S
NCHUNK = BPW // RPC
NPAIR = NCHUNK // 2

VP = V // 8
KP = 8 * H
NP = 8 * L
_PROJ_BLK = 1000
_PROJ_GRID = -(-VP // _PROJ_BLK)


def _project_body(tab_ref, whi_ref, wlo_ref, b_ref, o_ref):
    del wlo_ref
    t = tab_ref[...].reshape(_PROJ_BLK, KP)
    o_ref[...] = (
        jnp.dot(t, whi_ref[...], preferred_element_type=jnp.float32,
                precision=lax.Precision.HIGHEST)
        + b_ref[...]
    )


def _project(table8, whi, wlo, bblk):
    return pl.pallas_call(
        _project_body,
        grid=(_PROJ_GRID,),
        in_specs=[
            pl.BlockSpec((8 * _PROJ_BLK, H), lambda i: (i, 0)),
            pl.BlockSpec((KP, NP), lambda i: (0, 0)),
            pl.BlockSpec((KP, NP), lambda i: (0, 0)),
            pl.BlockSpec((1, NP), lambda i: (0, 0)),
        ],
        out_specs=pl.BlockSpec((_PROJ_BLK, NP), lambda i: (i, 0)),
        out_shape=jax.ShapeDtypeStruct((VP, NP), jnp.float32),
    )(table8, whi, wlo, bblk)


def _pool(proj, ids2d):
    mesh = plsc.VectorSubcoreMesh(core_axis_name="c", subcore_axis_name="s")

    @functools.partial(
        pl.kernel,
        out_type=jax.ShapeDtypeStruct((B, L), jnp.float32),
        mesh=mesh,
        scratch_types=[
            pltpu.VMEM((NCHUNK, CHUNK), jnp.int32),
            pltpu.VMEM((CHUNK, L), jnp.float32),
            pltpu.VMEM((CHUNK, L), jnp.float32),
            pltpu.VMEM((BPW, L), jnp.float32),
            pltpu.SemaphoreType.DMA,
            pltpu.SemaphoreType.DMA,
        ],
        compiler_params=pltpu.CompilerParams(use_tc_tiling_on_sc=False),
    )
    def k(proj_hbm, ids_hbm, out_hbm, idx_v, rows_a, rows_b, out_v, sem_a, sem_b):
        c = lax.axis_index("c")
        s = lax.axis_index("s")
        wid = c * NS + s

        pltpu.sync_copy(ids_hbm.at[pl.ds(wid * NCHUNK, NCHUNK)], idx_v)

        def g_start(ch, buf, sem):
            pltpu.async_copy(proj_hbm.at[idx_v.at[ch]], buf, sem)

        def g_wait(ch, buf, sem):
            pltpu.make_async_copy(proj_hbm.at[idx_v.at[ch]], buf, sem).wait()

        def reduce(ch, buf):
            for b in range(RPC):
                def body(i, a):
                    return a + buf[b * S + i]
                acc = lax.fori_loop(0, S, body, jnp.zeros((L,), jnp.float32),
                                    unroll=8)
                out_v[ch * RPC + b] = acc

        g_start(0, rows_a, sem_a)

        @pl.loop(0, NPAIR)
        def _pair(t):
            c0 = 2 * t
            c1 = c0 + 1
            g_wait(c0, rows_a, sem_a)
            g_start(c1, rows_b, sem_b)
            reduce(c0, rows_a)
            g_wait(c1, rows_b, sem_b)

            @pl.when(t < NPAIR - 1)
            def _():
                g_start(c0 + 2, rows_a, sem_a)

            reduce(c1, rows_b)

        pltpu.sync_copy(out_v, out_hbm.at[pl.ds(wid * BPW, BPW)])

    return k(proj, ids2d)


def kernel(input_ids, attention_mask, embedding_table, classifier_w, classifier_b):
    del attention_mask
    ids2d = input_ids.reshape(NW * NCHUNK, CHUNK).astype(jnp.int32)
    scale = jnp.float32(1.0 / S)
    wp = classifier_w.T * scale
    wblk = jnp.zeros((8, H, 8, L), jnp.float32)
    for i in range(8):
        wblk = wblk.at[i, :, i, :2].set(wp)
    wblk = wblk.reshape(KP, NP)
    whi = wblk
    wlo = wblk
    bblk = jnp.tile(
        jnp.zeros((L,), jnp.float32).at[:2].set(classifier_b * scale), 8
    ).reshape(1, NP)
    proj = _project(embedding_table, whi, wlo, bblk).reshape(V, L)
    pooled = _pool(proj, ids2d)
    return pooled[:, :2]


# speedup vs baseline: 1.5656x; 1.0575x over previous
import functools

import numpy as np
import jax
import jax.numpy as jnp
from jax import lax
from jax.experimental import pallas as pl
from jax.experimental.pallas import tpu as pltpu
from jax.experimental.pallas import tpu_sc as plsc

B = 4096
S = 200
V = 100000
H = 128
L = 16
NC = 2
NS = 16
NW = NC * NS
BPW = B // NW
IPW = BPW * S
GW = 128
NSLICE = IPW // GW

RPC = 8
CHUNK = RPC * S
NCHUNK = BPW // RPC
NPAIR = NCHUNK // 2

VP = V // 8
VPP = 12504
KP = 8 * H
NP = 8 * L
_PROJ_BLK = 1000
_PROJ_GRID = -(-VP // _PROJ_BLK)


def _split_hi(x):
    return lax.bitcast_convert_type(
        lax.bitcast_convert_type(x, jnp.int32) & jnp.int32(-65536), jnp.float32)


def _project_body(tab_ref, whi_ref, wlo_ref, b_ref, o_ref):
    t = tab_ref[...].reshape(_PROJ_BLK, KP)
    t1f = _split_hi(t)
    t1 = t1f.astype(jnp.bfloat16)
    t2 = (t - t1f).astype(jnp.bfloat16)
    dot = functools.partial(jnp.dot, preferred_element_type=jnp.float32)
    o_ref[...] = (dot(t1, whi_ref[...]) + dot(t1, wlo_ref[...])
                  + dot(t2, whi_ref[...]) + b_ref[...])


def _project(table8, whi, wlo, bblk):
    return pl.pallas_call(
        _project_body,
        grid=(_PROJ_GRID,),
        in_specs=[
            pl.BlockSpec((8 * _PROJ_BLK, H), lambda i: (i, 0)),
            pl.BlockSpec((KP, NP), lambda i: (0, 0)),
            pl.BlockSpec((KP, NP), lambda i: (0, 0)),
            pl.BlockSpec((1, NP), lambda i: (0, 0)),
        ],
        out_specs=pl.BlockSpec((_PROJ_BLK, NP), lambda i: (i, 0)),
        out_shape=jax.ShapeDtypeStruct((VPP, NP), jnp.float32),
    )(table8, whi, wlo, bblk)


def _pool(proj, ids2d):
    mesh = plsc.VectorSubcoreMesh(core_axis_name="c", subcore_axis_name="s")

    @functools.partial(
        pl.kernel,
        out_type=jax.ShapeDtypeStruct((B, L), jnp.float32),
        mesh=mesh,
        scratch_types=[
            pltpu.VMEM((NCHUNK, CHUNK), jnp.int32),
            pltpu.VMEM((CHUNK, L), jnp.float32),
            pltpu.VMEM((CHUNK, L), jnp.float32),
            pltpu.VMEM((BPW, L), jnp.float32),
            pltpu.SemaphoreType.DMA,
            pltpu.SemaphoreType.DMA,
        ],
        compiler_params=pltpu.CompilerParams(use_tc_tiling_on_sc=False),
    )
    def k(proj_hbm, ids_hbm, out_hbm, idx_v, rows_a, rows_b, out_v, sem_a, sem_b):
        c = lax.axis_index("c")
        s = lax.axis_index("s")
        wid = c * NS + s

        pltpu.sync_copy(ids_hbm.at[pl.ds(wid * NCHUNK, NCHUNK)], idx_v)

        def g_start(ch, buf, sem):
            pltpu.async_copy(proj_hbm.at[idx_v.at[ch]], buf, sem)

        def g_wait(ch, buf, sem):
            pltpu.make_async_copy(proj_hbm.at[idx_v.at[ch]], buf, sem).wait()

        def reduce(ch, buf):
            for b in range(RPC):
                def body(i, a):
                    return a + buf[b * S + i]
                acc = lax.fori_loop(0, S, body, jnp.zeros((L,), jnp.float32),
                                    unroll=8)
                out_v[ch * RPC + b] = acc

        g_start(0, rows_a, sem_a)

        @pl.loop(0, NPAIR)
        def _pair(t):
            c0 = 2 * t
            c1 = c0 + 1
            g_wait(c0, rows_a, sem_a)
            g_start(c1, rows_b, sem_b)
            reduce(c0, rows_a)
            g_wait(c1, rows_b, sem_b)

            @pl.when(t < NPAIR - 1)
            def _():
                g_start(c0 + 2, rows_a, sem_a)

            reduce(c1, rows_b)

        pltpu.sync_copy(out_v, out_hbm.at[pl.ds(wid * BPW, BPW)])

    return k(proj, ids2d)


def kernel(input_ids, attention_mask, embedding_table, classifier_w, classifier_b):
    del attention_mask
    ids2d = input_ids.reshape(NW * NCHUNK, CHUNK).astype(jnp.int32)
    scale = jnp.float32(1.0 / S)
    wp = classifier_w.T * scale
    wblk = jnp.zeros((8, H, 8, L), jnp.float32)
    for i in range(8):
        wblk = wblk.at[i, :, i, :2].set(wp)
    wblk = wblk.reshape(KP, NP)
    hi = lax.bitcast_convert_type(
        lax.bitcast_convert_type(wblk, jnp.int32) & jnp.int32(-65536),
        jnp.float32)
    lo = lax.bitcast_convert_type(
        lax.bitcast_convert_type(wblk - hi, jnp.int32) & jnp.int32(-65536),
        jnp.float32)
    whi = hi.astype(jnp.bfloat16)
    wlo = lo.astype(jnp.bfloat16)
    bblk = jnp.tile(
        jnp.zeros((L,), jnp.float32).at[:2].set(classifier_b * scale), 8
    ).reshape(1, NP)
    proj = _project(embedding_table, whi, wlo, bblk).reshape(8 * VPP, L)
    pooled = _pool(proj, ids2d)
    return pooled[:, :2]


# speedup vs baseline: 1.6953x; 1.0829x over previous
import functools

import numpy as np
import jax
import jax.numpy as jnp
from jax import lax
from jax.experimental import pallas as pl
from jax.experimental.pallas import tpu as pltpu
from jax.experimental.pallas import tpu_sc as plsc

B = 4096
S = 200
V = 100000
H = 128
L = 16
NC = 2
NS = 16
NW = NC * NS
BPW = B // NW
IPW = BPW * S
GW = 128
NSLICE = IPW // GW

RPC = 8
CHUNK = RPC * S
NCHUNK = BPW // RPC
NPAIR = NCHUNK // 2

VP = V // 8
VPP = 12504
KP = 8 * H
NP = 8 * L
_PROJ_BLK = 1000
_PROJ_GRID = -(-VP // _PROJ_BLK)


def _split_hi(x):
    return lax.bitcast_convert_type(
        lax.bitcast_convert_type(x, jnp.int32) & jnp.int32(-65536), jnp.float32)


def _project_body(tab_ref, whi_ref, wlo_ref, b_ref, o_ref):
    t = tab_ref[...]
    t1f = _split_hi(t)
    t1 = t1f.astype(jnp.bfloat16)
    t2 = (t - t1f).astype(jnp.bfloat16)
    dot = functools.partial(jnp.dot, preferred_element_type=jnp.float32)
    q = (dot(t1, whi_ref[...]) + dot(t1, wlo_ref[...])
         + dot(t2, whi_ref[...]) + b_ref[...])
    o_ref[:, :L] = q


def _project(table8, whi, wlo, bblk):
    return pl.pallas_call(
        _project_body,
        grid=(_PROJ_GRID,),
        in_specs=[
            pl.BlockSpec((8 * _PROJ_BLK, H), lambda i: (i, 0)),
            pl.BlockSpec((H, L), lambda i: (0, 0)),
            pl.BlockSpec((H, L), lambda i: (0, 0)),
            pl.BlockSpec((1, L), lambda i: (0, 0)),
        ],
        out_specs=pl.BlockSpec((8 * _PROJ_BLK, NP), lambda i: (i, 0)),
        out_shape=jax.ShapeDtypeStruct((V, NP), jnp.float32),
    )(table8, whi, wlo, bblk)


def _pool(proj, ids2d):
    mesh = plsc.VectorSubcoreMesh(core_axis_name="c", subcore_axis_name="s")

    @functools.partial(
        pl.kernel,
        out_type=jax.ShapeDtypeStruct((B, L), jnp.float32),
        mesh=mesh,
        scratch_types=[
            pltpu.VMEM((NCHUNK, CHUNK), jnp.int32),
            pltpu.VMEM((CHUNK, L), jnp.float32),
            pltpu.VMEM((CHUNK, L), jnp.float32),
            pltpu.VMEM((BPW, L), jnp.float32),
            pltpu.SemaphoreType.DMA,
            pltpu.SemaphoreType.DMA,
        ],
        compiler_params=pltpu.CompilerParams(use_tc_tiling_on_sc=False),
    )
    def k(proj_hbm, ids_hbm, out_hbm, idx_v, rows_a, rows_b, out_v, sem_a, sem_b):
        c = lax.axis_index("c")
        s = lax.axis_index("s")
        wid = c * NS + s

        pltpu.sync_copy(ids_hbm.at[pl.ds(wid * NCHUNK, NCHUNK)], idx_v)

        def g_start(ch, buf, sem):
            pltpu.async_copy(proj_hbm.at[idx_v.at[ch]], buf, sem)

        def g_wait(ch, buf, sem):
            pltpu.make_async_copy(proj_hbm.at[idx_v.at[ch]], buf, sem).wait()

        def reduce(ch, buf):
            for b in range(RPC):
                def body(i, a):
                    return a + buf[b * S + i]
                acc = lax.fori_loop(0, S, body, jnp.zeros((L,), jnp.float32),
                                    unroll=8)
                out_v[ch * RPC + b] = acc

        g_start(0, rows_a, sem_a)

        @pl.loop(0, NPAIR)
        def _pair(t):
            c0 = 2 * t
            c1 = c0 + 1
            g_wait(c0, rows_a, sem_a)
            g_start(c1, rows_b, sem_b)
            reduce(c0, rows_a)
            g_wait(c1, rows_b, sem_b)

            @pl.when(t < NPAIR - 1)
            def _():
                g_start(c0 + 2, rows_a, sem_a)

            reduce(c1, rows_b)

        pltpu.sync_copy(out_v, out_hbm.at[pl.ds(wid * BPW, BPW)])

    return k(proj, ids2d)


def kernel(input_ids, attention_mask, embedding_table, classifier_w, classifier_b):
    del attention_mask
    ids2d = input_ids.reshape(NW * NCHUNK, CHUNK).astype(jnp.int32) * 8
    scale = jnp.float32(1.0 / S)
    wpad = jnp.zeros((H, L), jnp.float32).at[:, :2].set(classifier_w.T * scale)
    hi = lax.bitcast_convert_type(
        lax.bitcast_convert_type(wpad, jnp.int32) & jnp.int32(-65536),
        jnp.float32)
    lo = lax.bitcast_convert_type(
        lax.bitcast_convert_type(wpad - hi, jnp.int32) & jnp.int32(-65536),
        jnp.float32)
    whi = hi.astype(jnp.bfloat16)
    wlo = lo.astype(jnp.bfloat16)
    bpad = jnp.zeros((1, L), jnp.float32).at[0, :2].set(classifier_b * scale)
    proj = _project(embedding_table, whi, wlo, bpad).reshape(8 * V, L)
    pooled = _pool(proj, ids2d)
    return pooled[:, :2]


# speedup vs baseline: 1.7582x; 1.0371x over previous
import functools

import numpy as np
import jax
import jax.numpy as jnp
from jax import lax
from jax.experimental import pallas as pl
from jax.experimental.pallas import tpu as pltpu
from jax.experimental.pallas import tpu_sc as plsc

B = 4096
S = 200
V = 100000
H = 128
L = 16
NC = 2
NS = 16
NW = NC * NS
BPW = B // NW
IPW = BPW * S
GW = 128
NSLICE = IPW // GW

RPC = 16
CHUNK = RPC * S
NCHUNK = BPW // RPC
NPAIR = NCHUNK // 2

VP = V // 8
VPP = 12504
KP = 8 * H
NP = 8 * L
_PROJ_BLK = 2000
_PROJ_GRID = -(-VP // _PROJ_BLK)


def _split_hi(x):
    return lax.bitcast_convert_type(
        lax.bitcast_convert_type(x, jnp.int32) & jnp.int32(-65536), jnp.float32)


def _project_body(tab_ref, whi_ref, wlo_ref, b_ref, o_ref):
    t = tab_ref[...]
    t1f = _split_hi(t)
    t1 = t1f.astype(jnp.bfloat16)
    t2 = (t - t1f).astype(jnp.bfloat16)
    dot = functools.partial(jnp.dot, preferred_element_type=jnp.float32)
    q = (dot(t1, whi_ref[...]) + dot(t1, wlo_ref[...])
         + dot(t2, whi_ref[...]) + b_ref[...])
    o_ref[:, :L] = q


def _project(table8, whi, wlo, bblk):
    return pl.pallas_call(
        _project_body,
        grid=(_PROJ_GRID,),
        in_specs=[
            pl.BlockSpec((8 * _PROJ_BLK, H), lambda i: (i, 0)),
            pl.BlockSpec((H, L), lambda i: (0, 0)),
            pl.BlockSpec((H, L), lambda i: (0, 0)),
            pl.BlockSpec((1, L), lambda i: (0, 0)),
        ],
        out_specs=pl.BlockSpec((8 * _PROJ_BLK, NP), lambda i: (i, 0)),
        out_shape=jax.ShapeDtypeStruct((V, NP), jnp.float32),
    )(table8, whi, wlo, bblk)


def _pool(proj, ids2d):
    mesh = plsc.VectorSubcoreMesh(core_axis_name="c", subcore_axis_name="s")

    @functools.partial(
        pl.kernel,
        out_type=jax.ShapeDtypeStruct((B, L), jnp.float32),
        mesh=mesh,
        scratch_types=[
            pltpu.VMEM((NCHUNK, CHUNK), jnp.int32),
            pltpu.VMEM((CHUNK, L), jnp.float32),
            pltpu.VMEM((CHUNK, L), jnp.float32),
            pltpu.VMEM((BPW, L), jnp.float32),
            pltpu.SemaphoreType.DMA,
            pltpu.SemaphoreType.DMA,
        ],
        compiler_params=pltpu.CompilerParams(use_tc_tiling_on_sc=False),
    )
    def k(proj_hbm, ids_hbm, out_hbm, idx_v, rows_a, rows_b, out_v, sem_a, sem_b):
        c = lax.axis_index("c")
        s = lax.axis_index("s")
        wid = c * NS + s

        pltpu.sync_copy(ids_hbm.at[pl.ds(wid * NCHUNK, NCHUNK)], idx_v)

        def g_start(ch, buf, sem):
            pltpu.async_copy(proj_hbm.at[idx_v.at[ch]], buf, sem)

        def g_wait(ch, buf, sem):
            pltpu.make_async_copy(proj_hbm.at[idx_v.at[ch]], buf, sem).wait()

        def reduce(ch, buf):
            for b in range(RPC):
                def body(i, a):
                    return a + buf[b * S + i]
                acc = lax.fori_loop(0, S, body, jnp.zeros((L,), jnp.float32),
                                    unroll=8)
                out_v[ch * RPC + b] = acc

        g_start(0, rows_a, sem_a)

        @pl.loop(0, NPAIR)
        def _pair(t):
            c0 = 2 * t
            c1 = c0 + 1
            g_wait(c0, rows_a, sem_a)
            g_start(c1, rows_b, sem_b)
            reduce(c0, rows_a)
            g_wait(c1, rows_b, sem_b)

            @pl.when(t < NPAIR - 1)
            def _():
                g_start(c0 + 2, rows_a, sem_a)

            reduce(c1, rows_b)

        pltpu.sync_copy(out_v, out_hbm.at[pl.ds(wid * BPW, BPW)])

    return k(proj, ids2d)


def kernel(input_ids, attention_mask, embedding_table, classifier_w, classifier_b):
    del attention_mask
    ids2d = input_ids.reshape(NW * NCHUNK, CHUNK).astype(jnp.int32) * 8
    scale = jnp.float32(1.0 / S)
    wpad = jnp.zeros((H, L), jnp.float32).at[:, :2].set(classifier_w.T * scale)
    hi = lax.bitcast_convert_type(
        lax.bitcast_convert_type(wpad, jnp.int32) & jnp.int32(-65536),
        jnp.float32)
    lo = lax.bitcast_convert_type(
        lax.bitcast_convert_type(wpad - hi, jnp.int32) & jnp.int32(-65536),
        jnp.float32)
    whi = hi.astype(jnp.bfloat16)
    wlo = lo.astype(jnp.bfloat16)
    bpad = jnp.zeros((1, L), jnp.float32).at[0, :2].set(classifier_b * scale)
    proj = _project(embedding_table, whi, wlo, bpad).reshape(8 * V, L)
    pooled = _pool(proj, ids2d)
    return pooled[:, :2]


# speedup vs baseline: 1.7820x; 1.0135x over previous
import functools

import numpy as np
import jax
import jax.numpy as jnp
from jax import lax
from jax.experimental import pallas as pl
from jax.experimental.pallas import tpu as pltpu
from jax.experimental.pallas import tpu_sc as plsc

B = 4096
S = 200
V = 100000
H = 128
L = 16
NC = 2
NS = 16
NW = NC * NS
BPW = B // NW
IPW = BPW * S
GW = 128
NSLICE = IPW // GW

RPC = 16
CHUNK = RPC * S
NCHUNK = BPW // RPC
NPAIR = NCHUNK // 2

VP = V // 8
VPP = 12504
KP = 8 * H
NP = 8 * L
_PROJ_BLK = 2000
_PROJ_GRID = -(-VP // _PROJ_BLK)


def _split_hi(x):
    return lax.bitcast_convert_type(
        lax.bitcast_convert_type(x, jnp.int32) & jnp.int32(-65536), jnp.float32)


def _project_body(tab_ref, whi_ref, wlo_ref, b_ref, o_ref):
    t = tab_ref[...]
    t1f = _split_hi(t)
    t1 = t1f.astype(jnp.bfloat16)
    t2 = (t - t1f).astype(jnp.bfloat16)
    dot = functools.partial(jnp.dot, preferred_element_type=jnp.float32)
    q = (dot(t1, whi_ref[...]) + dot(t1, wlo_ref[...])
         + dot(t2, whi_ref[...]) + b_ref[...])
    o_ref[:, :L] = q


def _project(table8, whi, wlo, bblk):
    return pl.pallas_call(
        _project_body,
        grid=(_PROJ_GRID,),
        in_specs=[
            pl.BlockSpec((8 * _PROJ_BLK, H), lambda i: (i, 0)),
            pl.BlockSpec((H, L), lambda i: (0, 0)),
            pl.BlockSpec((H, L), lambda i: (0, 0)),
            pl.BlockSpec((1, L), lambda i: (0, 0)),
        ],
        out_specs=pl.BlockSpec((8 * _PROJ_BLK, NP), lambda i: (i, 0)),
        out_shape=jax.ShapeDtypeStruct((V, NP), jnp.float32),
    )(table8, whi, wlo, bblk)


def _pool(proj, ids2d):
    mesh = plsc.VectorSubcoreMesh(core_axis_name="c", subcore_axis_name="s")

    @functools.partial(
        pl.kernel,
        out_type=jax.ShapeDtypeStruct((B, L), jnp.float32),
        mesh=mesh,
        scratch_types=[
            pltpu.VMEM((NCHUNK, CHUNK), jnp.int32),
            pltpu.VMEM((CHUNK, L), jnp.float32),
            pltpu.VMEM((CHUNK, L), jnp.float32),
            pltpu.VMEM((BPW, L), jnp.float32),
            pltpu.SemaphoreType.DMA,
            pltpu.SemaphoreType.DMA,
        ],
        compiler_params=pltpu.CompilerParams(use_tc_tiling_on_sc=False),
    )
    def k(proj_hbm, ids_hbm, out_hbm, idx_v, rows_a, rows_b, out_v, sem_a, sem_b):
        c = lax.axis_index("c")
        s = lax.axis_index("s")
        wid = c * NS + s

        pltpu.sync_copy(ids_hbm.at[pl.ds(wid * NCHUNK, NCHUNK)], idx_v)

        def g_start(ch, buf, sem):
            pltpu.async_copy(proj_hbm.at[idx_v.at[ch]], buf, sem)

        def g_wait(ch, buf, sem):
            pltpu.make_async_copy(proj_hbm.at[idx_v.at[ch]], buf, sem).wait()

        def reduce(ch, buf):
            for b in range(RPC):
                def body(i, accs):
                    base = b * S + i * 8
                    return tuple(a + buf[base + k] for k, a in enumerate(accs))
                accs = lax.fori_loop(
                    0, S // 8, body,
                    tuple(jnp.zeros((L,), jnp.float32) for _ in range(8)))
                acc = ((accs[0] + accs[1]) + (accs[2] + accs[3])) + (
                    (accs[4] + accs[5]) + (accs[6] + accs[7]))
                out_v[ch * RPC + b] = acc

        g_start(0, rows_a, sem_a)

        @pl.loop(0, NPAIR)
        def _pair(t):
            c0 = 2 * t
            c1 = c0 + 1
            g_wait(c0, rows_a, sem_a)
            g_start(c1, rows_b, sem_b)
            reduce(c0, rows_a)
            g_wait(c1, rows_b, sem_b)

            @pl.when(t < NPAIR - 1)
            def _():
                g_start(c0 + 2, rows_a, sem_a)

            reduce(c1, rows_b)

        pltpu.sync_copy(out_v, out_hbm.at[pl.ds(wid * BPW, BPW)])

    return k(proj, ids2d)


def kernel(input_ids, attention_mask, embedding_table, classifier_w, classifier_b):
    del attention_mask
    ids2d = input_ids.reshape(NW * NCHUNK, CHUNK).astype(jnp.int32) * 8
    scale = jnp.float32(1.0 / S)
    wpad = jnp.zeros((H, L), jnp.float32).at[:, :2].set(classifier_w.T * scale)
    hi = lax.bitcast_convert_type(
        lax.bitcast_convert_type(wpad, jnp.int32) & jnp.int32(-65536),
        jnp.float32)
    lo = lax.bitcast_convert_type(
        lax.bitcast_convert_type(wpad - hi, jnp.int32) & jnp.int32(-65536),
        jnp.float32)
    whi = hi.astype(jnp.bfloat16)
    wlo = lo.astype(jnp.bfloat16)
    bpad = jnp.zeros((1, L), jnp.float32).at[0, :2].set(classifier_b * scale)
    proj = _project(embedding_table, whi, wlo, bpad).reshape(8 * V, L)
    pooled = _pool(proj, ids2d)
    return pooled[:, :2]


# speedup vs baseline: 1.7850x; 1.0017x over previous
import functools

import jax
import jax.numpy as jnp
from jax import lax
from jax.experimental import pallas as pl
from jax.experimental.pallas import tpu as pltpu
from jax.experimental.pallas import tpu_sc as plsc

B = 4096
S = 200
V = 100000
H = 128
L = 16
NC = 2
NS = 16
NW = NC * NS
BPW = B // NW
RPC = 16
CHUNK = RPC * S
NCHUNK = BPW // RPC
NPAIR = NCHUNK // 2

NP = 128
VP = V // 8
_PROJ_BLK = 2000
_PROJ_GRID = -(-VP // _PROJ_BLK)


def _split_hi(x):
    return lax.bitcast_convert_type(
        lax.bitcast_convert_type(x, jnp.int32) & jnp.int32(-65536), jnp.float32)


def _project_body(tab_ref, whi_ref, wlo_ref, b_ref, o_ref):
    t = tab_ref[...]
    t1f = _split_hi(t)
    t1 = t1f.astype(jnp.bfloat16)
    t2 = (t - t1f).astype(jnp.bfloat16)
    dot = functools.partial(jnp.dot, preferred_element_type=jnp.float32)
    q = (dot(t1, whi_ref[...]) + dot(t1, wlo_ref[...])
         + dot(t2, whi_ref[...]) + b_ref[...])
    o_ref[:, :L] = q


def _project(table, whi, wlo, bpad):
    return pl.pallas_call(
        _project_body,
        grid=(_PROJ_GRID,),
        in_specs=[
            pl.BlockSpec((8 * _PROJ_BLK, H), lambda i: (i, 0)),
            pl.BlockSpec((H, L), lambda i: (0, 0)),
            pl.BlockSpec((H, L), lambda i: (0, 0)),
            pl.BlockSpec((1, L), lambda i: (0, 0)),
        ],
        out_specs=pl.BlockSpec((8 * _PROJ_BLK, NP), lambda i: (i, 0)),
        out_shape=jax.ShapeDtypeStruct((V, NP), jnp.float32),
    )(table, whi, wlo, bpad)


def _pool(proj, ids2d):
    mesh = plsc.VectorSubcoreMesh(core_axis_name="c", subcore_axis_name="s")

    @functools.partial(
        pl.kernel,
        out_type=jax.ShapeDtypeStruct((B, L), jnp.float32),
        mesh=mesh,
        scratch_types=[
            pltpu.VMEM((NCHUNK, CHUNK), jnp.int32),
            pltpu.VMEM((CHUNK, L), jnp.float32),
            pltpu.VMEM((CHUNK, L), jnp.float32),
            pltpu.VMEM((BPW, L), jnp.float32),
            pltpu.SemaphoreType.DMA,
            pltpu.SemaphoreType.DMA,
        ],
        compiler_params=pltpu.CompilerParams(use_tc_tiling_on_sc=False),
    )
    def k(proj_hbm, ids_hbm, out_hbm, idx_v, rows_a, rows_b, out_v, sem_a, sem_b):
        c = lax.axis_index("c")
        s = lax.axis_index("s")
        wid = c * NS + s

        pltpu.sync_copy(ids_hbm.at[pl.ds(wid * NCHUNK, NCHUNK)], idx_v)

        def g_start(ch, buf, sem):
            pltpu.async_copy(proj_hbm.at[idx_v.at[ch]], buf, sem)

        def g_wait(ch, buf, sem):
            pltpu.make_async_copy(proj_hbm.at[idx_v.at[ch]], buf, sem).wait()

        def reduce(ch, buf):
            for b in range(RPC):
                def body(i, accs):
                    base = b * S + i * 8
                    return tuple(a + buf[base + k] for k, a in enumerate(accs))
                accs = lax.fori_loop(
                    0, S // 8, body,
                    tuple(jnp.zeros((L,), jnp.float32) for _ in range(8)))
                acc = ((accs[0] + accs[1]) + (accs[2] + accs[3])) + (
                    (accs[4] + accs[5]) + (accs[6] + accs[7]))
                out_v[ch * RPC + b] = acc

        g_start(0, rows_a, sem_a)

        @pl.loop(0, NPAIR)
        def _pair(t):
            c0 = 2 * t
            c1 = c0 + 1
            g_wait(c0, rows_a, sem_a)
            g_start(c1, rows_b, sem_b)
            reduce(c0, rows_a)
            g_wait(c1, rows_b, sem_b)

            @pl.when(t < NPAIR - 1)
            def _():
                g_start(c0 + 2, rows_a, sem_a)

            reduce(c1, rows_b)

        pltpu.sync_copy(out_v, out_hbm.at[pl.ds(wid * BPW, BPW)])

    return k(proj, ids2d)


def kernel(input_ids, attention_mask, embedding_table, classifier_w, classifier_b):
    del attention_mask
    ids2d = input_ids.reshape(NW * NCHUNK, CHUNK).astype(jnp.int32) * 8
    scale = jnp.float32(1.0 / S)
    wpad = jnp.zeros((H, L), jnp.float32).at[:, :2].set(classifier_w.T * scale)
    hi = lax.bitcast_convert_type(
        lax.bitcast_convert_type(wpad, jnp.int32) & jnp.int32(-65536),
        jnp.float32)
    lo = lax.bitcast_convert_type(
        lax.bitcast_convert_type(wpad - hi, jnp.int32) & jnp.int32(-65536),
        jnp.float32)
    whi = hi.astype(jnp.bfloat16)
    wlo = lo.astype(jnp.bfloat16)
    bpad = jnp.zeros((1, L), jnp.float32).at[0, :2].set(classifier_b * scale)
    proj = _project(embedding_table, whi, wlo, bpad).reshape(8 * V, L)
    pooled = _pool(proj, ids2d)
    return pooled[:, :2]
